# Initial kernel scaffold; baseline (speedup 1.0000x reference)
import jax
import jax.numpy as jnp
from jax.experimental import pallas as pl


def kernel(h0, x, edges, edge_attr, node_mask, edge_mask, n_nodes, Wemb, bemb, We1, be1, We2, be2, Wn1, bn1, Wn2, bn2, Wd1, bd1, Wd2, bd2, Wg1, bg1, Wg2, bg2):
    raise NotImplementedError("write your pallas kernel here")


# speedup vs baseline: 1.2949x; 1.2949x over previous
import functools

import jax
import jax.numpy as jnp
from jax import lax
from jax.experimental import pallas as pl
from jax.experimental.pallas import tpu as pltpu
from jax.experimental.pallas import tpu_sc as plsc

N = 10000
E = 320000
HID = 256
NPG = 50
NL = 4

NC = 2
NS = 16
NW = NC * NS

BI = 128
NBLK = E // BI
HALF = N // NC
STRIPE = 320
HALF_PAD = NS * STRIPE
DUMMY = HALF_PAD - 1


def _silu(x):
    return x * jax.nn.sigmoid(x)


def _sc_gather_pair(tp, tq, row, col):
    D = tp.shape[1]
    iters = (NBLK + NW - 1) // NW

    @functools.partial(
        pl.kernel,
        out_type=(jax.ShapeDtypeStruct((E, D), jnp.float32),
                  jax.ShapeDtypeStruct((E, D), jnp.float32)),
        scratch_types=[pltpu.VMEM((BI,), jnp.int32),
                       pltpu.VMEM((BI,), jnp.int32),
                       pltpu.VMEM((BI, D), jnp.float32),
                       pltpu.VMEM((BI, D), jnp.float32),
                       pltpu.SemaphoreType.DMA],
        mesh=plsc.VectorSubcoreMesh(core_axis_name="c", subcore_axis_name="s"),
    )
    def k(tp_hbm, tq_hbm, row_hbm, col_hbm, gr_hbm, gc_hbm,
          idxr, idxc, pbuf, qbuf, sem):
        wid = lax.axis_index("s") * NC + lax.axis_index("c")

        def body(t, carry):
            j = wid + NW * t

            @pl.when(j < NBLK)
            def _():
                base = j * BI
                pltpu.sync_copy(row_hbm.at[pl.ds(base, BI)], idxr)
                pltpu.sync_copy(col_hbm.at[pl.ds(base, BI)], idxc)
                c1 = pltpu.async_copy(tp_hbm.at[idxr], pbuf, sem)
                c2 = pltpu.async_copy(tq_hbm.at[idxc], qbuf, sem)
                c1.wait()
                c2.wait()
                pltpu.sync_copy(pbuf, gr_hbm.at[pl.ds(base, BI)])
                pltpu.sync_copy(qbuf, gc_hbm.at[pl.ds(base, BI)])

            return carry

        lax.fori_loop(0, iters, body, 0)

    return k(tp, tq, row, col)


SHARD = 320
LAST_SHARD = N - (NW - 1) * SHARD


def _sc_scatter_sorted(ef, perm_pad, rowp_pad, bounds):
    @functools.partial(
        pl.kernel,
        out_type=jax.ShapeDtypeStruct((N, HID), jnp.float32),
        scratch_types=[pltpu.VMEM((56,), jnp.int32),
                       pltpu.VMEM((BI,), jnp.int32),
                       pltpu.VMEM((BI + 16,), jnp.int32),
                       pltpu.VMEM((BI, HID), jnp.float32),
                       pltpu.VMEM((SHARD, HID), jnp.float32),
                       pltpu.SemaphoreType.DMA],
        mesh=plsc.VectorSubcoreMesh(core_axis_name="c", subcore_axis_name="s"),
    )
    def k(ef_hbm, perm_hbm, rowp_hbm, bounds_hbm, out_hbm,
          bvuf, pbufi, rbuf, efb, acc, sem):
        wid = lax.axis_index("s") * NC + lax.axis_index("c")
        pltpu.sync_copy(bounds_hbm, bvuf.at[pl.ds(0, 40)])
        lo = bvuf[pl.ds(wid, 16)][0]
        hi = bvuf[pl.ds(wid + 1, 16)][0]
        base = wid * SHARD

        def zr(r, carry):
            for c in range(HID // 16):
                acc[r, pl.ds(c * 16, 16)] = jnp.zeros((16,), jnp.float32)
            return carry

        lax.fori_loop(0, SHARD, zr, 0)

        jb0 = (lo // BI) * BI

        def body(t, carry):
            jb = pl.multiple_of(jb0 + t * BI, BI)

            @pl.when(jb < hi)
            def _():
                pltpu.sync_copy(perm_hbm.at[pl.ds(jb, BI)], pbufi)
                pltpu.sync_copy(rowp_hbm.at[pl.ds(jb, BI)], rbuf.at[pl.ds(0, BI)])
                pltpu.async_copy(ef_hbm.at[pbufi], efb, sem).wait()

                def rbody(k2, carry2):
                    rid = rbuf[pl.ds(k2, 16)][0]
                    loc = rid - base
                    ok = (jb + k2 < hi) & (loc >= 0) & (loc < SHARD)

                    @pl.when(ok)
                    def _():
                        for c in range(HID // 16):
                            plsc.addupdate(acc.at[loc, pl.ds(c * 16, 16)],
                                           efb[k2, pl.ds(c * 16, 16)])

                    return carry2

                lax.fori_loop(0, BI, rbody, 0)

            return carry

        lax.fori_loop(0, NBLK, body, 0)

        @pl.when(wid < NW - 1)
        def _():
            pltpu.sync_copy(acc.at[pl.ds(0, SHARD)],
                            out_hbm.at[pl.ds(base, SHARD)])

        @pl.when(wid == NW - 1)
        def _():
            pltpu.sync_copy(acc.at[pl.ds(0, LAST_SHARD)],
                            out_hbm.at[pl.ds(base, LAST_SHARD)])

    return k(ef, perm_pad, rowp_pad, bounds)


def _tc_emb(h0, Wemb, bemb):
    BN = 2000

    def body(x_ref, w_ref, b_ref, o_ref):
        o_ref[...] = (jnp.dot(x_ref[...], w_ref[...],
                              preferred_element_type=jnp.float32) + b_ref[...])

    return pl.pallas_call(
        body,
        grid=(N // BN,),
        in_specs=[pl.BlockSpec((BN, h0.shape[1]), lambda i: (i, 0)),
                  pl.BlockSpec(Wemb.shape, lambda i: (0, 0)),
                  pl.BlockSpec((1, HID), lambda i: (0, 0))],
        out_specs=pl.BlockSpec((BN, HID), lambda i: (i, 0)),
        out_shape=jax.ShapeDtypeStruct((N, HID), jnp.float32),
    )(h0, Wemb, bemb)


def _tc_pq(h, A, B):
    BN = 1000

    def body(h_ref, a_ref, b_ref, p_ref, q_ref):
        hb = h_ref[...]
        p_ref[...] = jnp.dot(hb, a_ref[...], preferred_element_type=jnp.float32)
        q_ref[...] = jnp.dot(hb, b_ref[...], preferred_element_type=jnp.float32)

    return pl.pallas_call(
        body,
        grid=(N // BN,),
        in_specs=[pl.BlockSpec((BN, HID), lambda i: (i, 0)),
                  pl.BlockSpec((HID, HID), lambda i: (0, 0)),
                  pl.BlockSpec((HID, HID), lambda i: (0, 0))],
        out_specs=[pl.BlockSpec((BN, HID), lambda i: (i, 0)),
                   pl.BlockSpec((BN, HID), lambda i: (i, 0))],
        out_shape=[jax.ShapeDtypeStruct((N, HID), jnp.float32),
                   jax.ShapeDtypeStruct((N, HID), jnp.float32)],
    )(h, A, B)


def _tc_radial(xr, xc):
    BE = 2000
    D = xr.shape[1]

    def body(r_ref, c_ref, o_ref):
        d = r_ref[...] - c_ref[...]
        o_ref[...] = jnp.sum(d * d, axis=1, keepdims=True)

    return pl.pallas_call(
        body,
        grid=(E // BE,),
        in_specs=[pl.BlockSpec((BE, D), lambda i: (i, 0)),
                  pl.BlockSpec((BE, D), lambda i: (i, 0))],
        out_specs=pl.BlockSpec((BE, 1), lambda i: (i, 0)),
        out_shape=jax.ShapeDtypeStruct((E, 1), jnp.float32),
    )(xr, xc)


def _tc_edge(Gr, Gc, radial, attr, mask, wr, C, be1, We2, be2):
    BE = 512

    def body(gr_ref, gc_ref, rad_ref, at_ref, mk_ref,
             wr_ref, c_ref, b1_ref, w2_ref, b2_ref, o_ref):
        t = gr_ref[...] + gc_ref[...] + rad_ref[...] * wr_ref[...] + b1_ref[...]
        at = at_ref[...]
        cw = c_ref[...]
        for k in range(4):
            t += at[:, k:k + 1] * cw[k:k + 1, :]
        m = _silu(t)
        z = jnp.dot(m, w2_ref[...], preferred_element_type=jnp.float32) + b2_ref[...]
        o_ref[...] = _silu(z) * mk_ref[...]

    return pl.pallas_call(
        body,
        grid=(E // BE,),
        in_specs=[pl.BlockSpec((BE, HID), lambda i: (i, 0)),
                  pl.BlockSpec((BE, HID), lambda i: (i, 0)),
                  pl.BlockSpec((BE, 1), lambda i: (i, 0)),
                  pl.BlockSpec((BE, 4), lambda i: (i, 0)),
                  pl.BlockSpec((BE, 1), lambda i: (i, 0)),
                  pl.BlockSpec((1, HID), lambda i: (0, 0)),
                  pl.BlockSpec((4, HID), lambda i: (0, 0)),
                  pl.BlockSpec((1, HID), lambda i: (0, 0)),
                  pl.BlockSpec((HID, HID), lambda i: (0, 0)),
                  pl.BlockSpec((1, HID), lambda i: (0, 0))],
        out_specs=pl.BlockSpec((BE, HID), lambda i: (i, 0)),
        out_shape=jax.ShapeDtypeStruct((E, HID), jnp.float32),
    )(Gr, Gc, radial, attr, mask, wr, C, be1, We2, be2)


def _tc_node(h, agg, h0, Wh, Wa, W0, bn1, Wn2, bn2):
    BN = 1000

    def body(h_ref, a_ref, h0_ref, wh_ref, wa_ref, w0_ref,
             b1_ref, w2_ref, b2_ref, o_ref):
        hb = h_ref[...]
        t = (jnp.dot(hb, wh_ref[...], preferred_element_type=jnp.float32)
             + jnp.dot(a_ref[...], wa_ref[...], preferred_element_type=jnp.float32)
             + jnp.dot(h0_ref[...], w0_ref[...], preferred_element_type=jnp.float32)
             + b1_ref[...])
        u = jnp.dot(_silu(t), w2_ref[...], preferred_element_type=jnp.float32) + b2_ref[...]
        o_ref[...] = hb + u

    return pl.pallas_call(
        body,
        grid=(N // BN,),
        in_specs=[pl.BlockSpec((BN, HID), lambda i: (i, 0)),
                  pl.BlockSpec((BN, HID), lambda i: (i, 0)),
                  pl.BlockSpec((BN, h0.shape[1]), lambda i: (i, 0)),
                  pl.BlockSpec((HID, HID), lambda i: (0, 0)),
                  pl.BlockSpec((HID, HID), lambda i: (0, 0)),
                  pl.BlockSpec((h0.shape[1], HID), lambda i: (0, 0)),
                  pl.BlockSpec((1, HID), lambda i: (0, 0)),
                  pl.BlockSpec((HID, HID), lambda i: (0, 0)),
                  pl.BlockSpec((1, HID), lambda i: (0, 0))],
        out_specs=pl.BlockSpec((BN, HID), lambda i: (i, 0)),
        out_shape=jax.ShapeDtypeStruct((N, HID), jnp.float32),
    )(h, agg, h0, Wh, Wa, W0, bn1, Wn2, bn2)


def _tc_dec_pool(h, mask, Wd1, bd1, Wd2, bd2):
    BN = 1000
    GPB = BN // NPG
    NG = N // NPG

    def body(h_ref, mk_ref, w1_ref, b1_ref, w2_ref, b2_ref, o_ref):
        t = _silu(jnp.dot(h_ref[...], w1_ref[...],
                          preferred_element_type=jnp.float32) + b1_ref[...])
        hd = (jnp.dot(t, w2_ref[...], preferred_element_type=jnp.float32)
              + b2_ref[...]) * mk_ref[...]
        for g in range(GPB):
            o_ref[0, g, :] = jnp.sum(hd[g * NPG:(g + 1) * NPG, :], axis=0)

    out = pl.pallas_call(
        body,
        grid=(N // BN,),
        in_specs=[pl.BlockSpec((BN, HID), lambda i: (i, 0)),
                  pl.BlockSpec((BN, 1), lambda i: (i, 0)),
                  pl.BlockSpec((HID, HID), lambda i: (0, 0)),
                  pl.BlockSpec((1, HID), lambda i: (0, 0)),
                  pl.BlockSpec((HID, HID), lambda i: (0, 0)),
                  pl.BlockSpec((1, HID), lambda i: (0, 0))],
        out_specs=pl.BlockSpec((1, GPB, HID), lambda i: (i, 0, 0)),
        out_shape=jax.ShapeDtypeStruct((N // BN, GPB, HID), jnp.float32),
    )(h, mask, Wd1, bd1, Wd2, bd2)
    return out.reshape(NG, HID)


def _tc_head(hg, Wg1, bg1, Wg2, bg2):
    NG = hg.shape[0]

    def body(h_ref, w1_ref, b1_ref, w2_ref, b2_ref, o_ref):
        t = _silu(jnp.dot(h_ref[...], w1_ref[...],
                          preferred_element_type=jnp.float32) + b1_ref[...])
        o_ref[...] = jnp.dot(t, w2_ref[...],
                             preferred_element_type=jnp.float32) + b2_ref[...]

    return pl.pallas_call(
        body,
        grid=(1,),
        in_specs=[pl.BlockSpec((NG, HID), lambda i: (0, 0)),
                  pl.BlockSpec((HID, HID), lambda i: (0, 0)),
                  pl.BlockSpec((1, HID), lambda i: (0, 0)),
                  pl.BlockSpec((HID, 1), lambda i: (0, 0)),
                  pl.BlockSpec((1, 1), lambda i: (0, 0))],
        out_specs=pl.BlockSpec((NG, 1), lambda i: (0, 0)),
        out_shape=jax.ShapeDtypeStruct((NG, 1), jnp.float32),
    )(hg, Wg1, bg1, Wg2, bg2)


def kernel(h0, x, edges, edge_attr, node_mask, edge_mask, n_nodes,
           Wemb, bemb, We1, be1, We2, be2, Wn1, bn1, Wn2, bn2,
           Wd1, bd1, Wd2, bd2, Wg1, bg1, Wg2, bg2):
    row = edges[0]
    col = edges[1]
    xpad = jnp.pad(x, ((0, 0), (0, 128 - x.shape[1])))

    order = jnp.argsort(row).astype(jnp.int32)
    rows_sorted = row[order]
    bounds = jnp.searchsorted(
        rows_sorted, jnp.arange(NW + 1, dtype=jnp.int32) * SHARD).astype(jnp.int32)
    bounds = jnp.pad(bounds, (0, 40 - NW - 1))
    perm_pad = jnp.pad(order, (0, BI))
    rowp_pad = jnp.pad(rows_sorted, (0, BI))

    h = _tc_emb(h0, Wemb, bemb.reshape(1, -1))
    xr, xc = _sc_gather_pair(xpad, xpad, row, col)
    radial = _tc_radial(xr, xc)

    for i in range(NL):
        A = We1[i, :HID]
        B = We1[i, HID:2 * HID]
        wr = We1[i, 2 * HID:2 * HID + 1]
        C = We1[i, 2 * HID + 1:]
        P, Q = _tc_pq(h, A, B)
        Gr, Gc = _sc_gather_pair(P, Q, row, col)
        ef = _tc_edge(Gr, Gc, radial, edge_attr, edge_mask,
                      wr, C, be1[i].reshape(1, -1), We2[i], be2[i].reshape(1, -1))
        agg = _sc_scatter_sorted(ef, perm_pad, rowp_pad, bounds)
        h = _tc_node(h, agg, h0, Wn1[i, :HID], Wn1[i, HID:2 * HID],
                     Wn1[i, 2 * HID:], bn1[i].reshape(1, -1), Wn2[i],
                     bn2[i].reshape(1, -1))

    hg = _tc_dec_pool(h, node_mask, Wd1, bd1.reshape(1, -1), Wd2,
                      bd2.reshape(1, -1))
    pred = _tc_head(hg, Wg1, bg1.reshape(1, -1), Wg2, bg2.reshape(1, -1))
    return pred[:, 0]


# speedup vs baseline: 1.3689x; 1.0572x over previous
import functools

import jax
import jax.numpy as jnp
from jax import lax
from jax.experimental import pallas as pl
from jax.experimental.pallas import tpu as pltpu
from jax.experimental.pallas import tpu_sc as plsc

N = 10000
E = 320000
HID = 256
NPG = 50
NL = 4

NC = 2
NS = 16
NW = NC * NS

BI = 128
NBLK = E // BI
HALF = N // NC
STRIPE = 320
HALF_PAD = NS * STRIPE
DUMMY = HALF_PAD - 1


def _silu(x):
    return x * jax.nn.sigmoid(x)


def _sc_gather_pair(tp, tq, row, col):
    D = tp.shape[1]
    iters = (NBLK + NW - 1) // NW

    @functools.partial(
        pl.kernel,
        out_type=(jax.ShapeDtypeStruct((E, D), jnp.float32),
                  jax.ShapeDtypeStruct((E, D), jnp.float32)),
        scratch_types=[pltpu.VMEM((BI,), jnp.int32),
                       pltpu.VMEM((BI,), jnp.int32),
                       pltpu.VMEM((BI, D), jnp.float32),
                       pltpu.VMEM((BI, D), jnp.float32),
                       pltpu.SemaphoreType.DMA],
        mesh=plsc.VectorSubcoreMesh(core_axis_name="c", subcore_axis_name="s"),
    )
    def k(tp_hbm, tq_hbm, row_hbm, col_hbm, gr_hbm, gc_hbm,
          idxr, idxc, pbuf, qbuf, sem):
        wid = lax.axis_index("s") * NC + lax.axis_index("c")

        def body(t, carry):
            j = wid + NW * t

            @pl.when(j < NBLK)
            def _():
                base = j * BI
                pltpu.sync_copy(row_hbm.at[pl.ds(base, BI)], idxr)
                pltpu.sync_copy(col_hbm.at[pl.ds(base, BI)], idxc)
                c1 = pltpu.async_copy(tp_hbm.at[idxr], pbuf, sem)
                c2 = pltpu.async_copy(tq_hbm.at[idxc], qbuf, sem)
                c1.wait()
                c2.wait()
                pltpu.sync_copy(pbuf, gr_hbm.at[pl.ds(base, BI)])
                pltpu.sync_copy(qbuf, gc_hbm.at[pl.ds(base, BI)])

            return carry

        lax.fori_loop(0, iters, body, 0)

    return k(tp, tq, row, col)


SHARD = 320
LAST_SHARD = N - (NW - 1) * SHARD


def _sc_scatter_sorted(ef, perm_pad, rowp_pad, bounds):
    @functools.partial(
        pl.kernel,
        out_type=jax.ShapeDtypeStruct((N, HID), jnp.float32),
        scratch_types=[pltpu.VMEM((56,), jnp.int32),
                       pltpu.VMEM((BI,), jnp.int32),
                       pltpu.VMEM((BI + 16,), jnp.int32),
                       pltpu.VMEM((BI, HID), jnp.float32),
                       pltpu.VMEM((SHARD + 8, HID), jnp.float32),
                       pltpu.SemaphoreType.DMA],
        mesh=plsc.VectorSubcoreMesh(core_axis_name="c", subcore_axis_name="s"),
    )
    def k(ef_hbm, perm_hbm, rowp_hbm, bounds_hbm, out_hbm,
          bvuf, pbufi, rbuf, efb, acc, sem):
        wid = lax.axis_index("s") * NC + lax.axis_index("c")
        pltpu.sync_copy(bounds_hbm, bvuf.at[pl.ds(0, 40)])
        lo = bvuf[pl.ds(wid, 16)][0]
        hi = bvuf[pl.ds(wid + 1, 16)][0]
        base = wid * SHARD

        def zr(r, carry):
            for c in range(HID // 16):
                acc[r, pl.ds(c * 16, 16)] = jnp.zeros((16,), jnp.float32)
            return carry

        lax.fori_loop(0, SHARD + 8, zr, 0)

        jb0 = (lo // BI) * BI

        def body(t, carry):
            jb = pl.multiple_of(jb0 + t * BI, BI)

            @pl.when(jb < hi)
            def _():
                pltpu.sync_copy(perm_hbm.at[pl.ds(jb, BI)], pbufi)
                pltpu.sync_copy(rowp_hbm.at[pl.ds(jb, BI)], rbuf.at[pl.ds(0, BI)])
                pltpu.async_copy(ef_hbm.at[pbufi], efb, sem).wait()

                def rbody(k2, carry2):
                    rid = rbuf[pl.ds(k2, 16)][0]
                    loc = rid - base
                    ok = (jb + k2 < hi) & (loc >= 0) & (loc < SHARD)
                    loc = jnp.where(ok, loc, SHARD)
                    for c in range(HID // 16):
                        plsc.addupdate(acc.at[loc, pl.ds(c * 16, 16)],
                                       efb[k2, pl.ds(c * 16, 16)])
                    return carry2

                lax.fori_loop(0, BI, rbody, 0)

            return carry

        lax.fori_loop(0, NBLK, body, 0)

        @pl.when(wid < NW - 1)
        def _():
            pltpu.sync_copy(acc.at[pl.ds(0, SHARD)],
                            out_hbm.at[pl.ds(base, SHARD)])

        @pl.when(wid == NW - 1)
        def _():
            pltpu.sync_copy(acc.at[pl.ds(0, LAST_SHARD)],
                            out_hbm.at[pl.ds(base, LAST_SHARD)])

    return k(ef, perm_pad, rowp_pad, bounds)


def _tc_emb(h0, Wemb, bemb):
    BN = 2000

    def body(x_ref, w_ref, b_ref, o_ref):
        o_ref[...] = (jnp.dot(x_ref[...], w_ref[...],
                              preferred_element_type=jnp.float32) + b_ref[...])

    return pl.pallas_call(
        body,
        grid=(N // BN,),
        in_specs=[pl.BlockSpec((BN, h0.shape[1]), lambda i: (i, 0)),
                  pl.BlockSpec(Wemb.shape, lambda i: (0, 0)),
                  pl.BlockSpec((1, HID), lambda i: (0, 0))],
        out_specs=pl.BlockSpec((BN, HID), lambda i: (i, 0)),
        out_shape=jax.ShapeDtypeStruct((N, HID), jnp.float32),
    )(h0, Wemb, bemb)


def _tc_pq(h, A, B):
    BN = 1000

    def body(h_ref, a_ref, b_ref, p_ref, q_ref):
        hb = h_ref[...]
        p = jnp.dot(hb, a_ref[...], preferred_element_type=jnp.float32)
        q = jnp.dot(hb, b_ref[...], preferred_element_type=jnp.float32)
        p_ref[...] = p.astype(jnp.bfloat16)
        q_ref[...] = q.astype(jnp.bfloat16)

    P, Q = pl.pallas_call(
        body,
        grid=(N // BN,),
        in_specs=[pl.BlockSpec((BN, HID), lambda i: (i, 0)),
                  pl.BlockSpec((HID, HID), lambda i: (0, 0)),
                  pl.BlockSpec((HID, HID), lambda i: (0, 0))],
        out_specs=[pl.BlockSpec((BN, HID), lambda i: (i, 0)),
                   pl.BlockSpec((BN, HID), lambda i: (i, 0))],
        out_shape=[jax.ShapeDtypeStruct((N, HID), jnp.bfloat16),
                   jax.ShapeDtypeStruct((N, HID), jnp.bfloat16)],
    )(h, A, B)
    Ppk = jax.lax.bitcast_convert_type(P.reshape(N, HID // 2, 2), jnp.float32)
    Qpk = jax.lax.bitcast_convert_type(Q.reshape(N, HID // 2, 2), jnp.float32)
    return Ppk, Qpk


def _tc_radial(xr, xc):
    BE = 2000
    D = xr.shape[1]

    def body(r_ref, c_ref, o_ref):
        d = r_ref[...] - c_ref[...]
        o_ref[...] = jnp.sum(d * d, axis=1, keepdims=True)

    return pl.pallas_call(
        body,
        grid=(E // BE,),
        in_specs=[pl.BlockSpec((BE, D), lambda i: (i, 0)),
                  pl.BlockSpec((BE, D), lambda i: (i, 0))],
        out_specs=pl.BlockSpec((BE, 1), lambda i: (i, 0)),
        out_shape=jax.ShapeDtypeStruct((E, 1), jnp.float32),
    )(xr, xc)


def _unpack_planes(packed_f32):
    u = jax.lax.bitcast_convert_type(packed_f32, jnp.int32)
    even = jax.lax.bitcast_convert_type(u << 16, jnp.float32)
    odd = jax.lax.bitcast_convert_type(u & jnp.int32(-65536), jnp.float32)
    return jnp.concatenate([even, odd], axis=1)


def _tc_edge(Gr, Gc, radial, attr, mask, wr_p, C_p, be1_p, We2_p, be2):
    BE = 512

    def body(gr_ref, gc_ref, rad_ref, at_ref, mk_ref,
             wr_ref, c_ref, b1_ref, w2_ref, b2_ref, o_ref):
        t = (_unpack_planes(gr_ref[...]) + _unpack_planes(gc_ref[...])
             + rad_ref[...] * wr_ref[...] + b1_ref[...])
        at = at_ref[...]
        cw = c_ref[...]
        for k in range(4):
            t += at[:, k:k + 1] * cw[k:k + 1, :]
        m = _silu(t)
        z = jnp.dot(m.astype(jnp.bfloat16), w2_ref[...],
                    preferred_element_type=jnp.float32) + b2_ref[...]
        o_ref[...] = _silu(z) * mk_ref[...]

    return pl.pallas_call(
        body,
        grid=(E // BE,),
        in_specs=[pl.BlockSpec((BE, HID // 2), lambda i: (i, 0)),
                  pl.BlockSpec((BE, HID // 2), lambda i: (i, 0)),
                  pl.BlockSpec((BE, 1), lambda i: (i, 0)),
                  pl.BlockSpec((BE, 4), lambda i: (i, 0)),
                  pl.BlockSpec((BE, 1), lambda i: (i, 0)),
                  pl.BlockSpec((1, HID), lambda i: (0, 0)),
                  pl.BlockSpec((4, HID), lambda i: (0, 0)),
                  pl.BlockSpec((1, HID), lambda i: (0, 0)),
                  pl.BlockSpec((HID, HID), lambda i: (0, 0)),
                  pl.BlockSpec((1, HID), lambda i: (0, 0))],
        out_specs=pl.BlockSpec((BE, HID), lambda i: (i, 0)),
        out_shape=jax.ShapeDtypeStruct((E, HID), jnp.float32),
    )(Gr, Gc, radial, attr, mask, wr_p, C_p, be1_p, We2_p, be2)


def _tc_node(h, agg, h0, Wh, Wa, W0, bn1, Wn2, bn2):
    BN = 1000

    def body(h_ref, a_ref, h0_ref, wh_ref, wa_ref, w0_ref,
             b1_ref, w2_ref, b2_ref, o_ref):
        hb = h_ref[...]
        t = (jnp.dot(hb, wh_ref[...], preferred_element_type=jnp.float32)
             + jnp.dot(a_ref[...], wa_ref[...], preferred_element_type=jnp.float32)
             + jnp.dot(h0_ref[...], w0_ref[...], preferred_element_type=jnp.float32)
             + b1_ref[...])
        u = jnp.dot(_silu(t), w2_ref[...], preferred_element_type=jnp.float32) + b2_ref[...]
        o_ref[...] = hb + u

    return pl.pallas_call(
        body,
        grid=(N // BN,),
        in_specs=[pl.BlockSpec((BN, HID), lambda i: (i, 0)),
                  pl.BlockSpec((BN, HID), lambda i: (i, 0)),
                  pl.BlockSpec((BN, h0.shape[1]), lambda i: (i, 0)),
                  pl.BlockSpec((HID, HID), lambda i: (0, 0)),
                  pl.BlockSpec((HID, HID), lambda i: (0, 0)),
                  pl.BlockSpec((h0.shape[1], HID), lambda i: (0, 0)),
                  pl.BlockSpec((1, HID), lambda i: (0, 0)),
                  pl.BlockSpec((HID, HID), lambda i: (0, 0)),
                  pl.BlockSpec((1, HID), lambda i: (0, 0))],
        out_specs=pl.BlockSpec((BN, HID), lambda i: (i, 0)),
        out_shape=jax.ShapeDtypeStruct((N, HID), jnp.float32),
    )(h, agg, h0, Wh, Wa, W0, bn1, Wn2, bn2)


def _tc_dec_pool(h, mask, Wd1, bd1, Wd2, bd2):
    BN = 1000
    GPB = BN // NPG
    NG = N // NPG

    def body(h_ref, mk_ref, w1_ref, b1_ref, w2_ref, b2_ref, o_ref):
        t = _silu(jnp.dot(h_ref[...], w1_ref[...],
                          preferred_element_type=jnp.float32) + b1_ref[...])
        hd = (jnp.dot(t, w2_ref[...], preferred_element_type=jnp.float32)
              + b2_ref[...]) * mk_ref[...]
        for g in range(GPB):
            o_ref[0, g, :] = jnp.sum(hd[g * NPG:(g + 1) * NPG, :], axis=0)

    out = pl.pallas_call(
        body,
        grid=(N // BN,),
        in_specs=[pl.BlockSpec((BN, HID), lambda i: (i, 0)),
                  pl.BlockSpec((BN, 1), lambda i: (i, 0)),
                  pl.BlockSpec((HID, HID), lambda i: (0, 0)),
                  pl.BlockSpec((1, HID), lambda i: (0, 0)),
                  pl.BlockSpec((HID, HID), lambda i: (0, 0)),
                  pl.BlockSpec((1, HID), lambda i: (0, 0))],
        out_specs=pl.BlockSpec((1, GPB, HID), lambda i: (i, 0, 0)),
        out_shape=jax.ShapeDtypeStruct((N // BN, GPB, HID), jnp.float32),
    )(h, mask, Wd1, bd1, Wd2, bd2)
    return out.reshape(NG, HID)


def _tc_head(hg, Wg1, bg1, Wg2, bg2):
    NG = hg.shape[0]

    def body(h_ref, w1_ref, b1_ref, w2_ref, b2_ref, o_ref):
        t = _silu(jnp.dot(h_ref[...], w1_ref[...],
                          preferred_element_type=jnp.float32) + b1_ref[...])
        o_ref[...] = jnp.dot(t, w2_ref[...],
                             preferred_element_type=jnp.float32) + b2_ref[...]

    return pl.pallas_call(
        body,
        grid=(1,),
        in_specs=[pl.BlockSpec((NG, HID), lambda i: (0, 0)),
                  pl.BlockSpec((HID, HID), lambda i: (0, 0)),
                  pl.BlockSpec((1, HID), lambda i: (0, 0)),
                  pl.BlockSpec((HID, 1), lambda i: (0, 0)),
                  pl.BlockSpec((1, 1), lambda i: (0, 0))],
        out_specs=pl.BlockSpec((NG, 1), lambda i: (0, 0)),
        out_shape=jax.ShapeDtypeStruct((NG, 1), jnp.float32),
    )(hg, Wg1, bg1, Wg2, bg2)


def kernel(h0, x, edges, edge_attr, node_mask, edge_mask, n_nodes,
           Wemb, bemb, We1, be1, We2, be2, Wn1, bn1, Wn2, bn2,
           Wd1, bd1, Wd2, bd2, Wg1, bg1, Wg2, bg2):
    row = edges[0]
    col = edges[1]
    xpad = jnp.pad(x, ((0, 0), (0, 128 - x.shape[1])))

    order = jnp.argsort(row).astype(jnp.int32)
    rows_sorted = row[order]
    bounds = jnp.searchsorted(
        rows_sorted, jnp.arange(NW + 1, dtype=jnp.int32) * SHARD).astype(jnp.int32)
    bounds = jnp.pad(bounds, (0, 40 - NW - 1))
    perm_pad = jnp.pad(order, (0, BI))
    rowp_pad = jnp.pad(rows_sorted, (0, BI))

    h = _tc_emb(h0, Wemb, bemb.reshape(1, -1))
    xr, xc = _sc_gather_pair(xpad, xpad, row, col)
    radial = _tc_radial(xr, xc)

    plane = jnp.concatenate([jnp.arange(0, HID, 2), jnp.arange(1, HID, 2)])

    for i in range(NL):
        A = We1[i, :HID]
        B = We1[i, HID:2 * HID]
        wr = We1[i, 2 * HID:2 * HID + 1]
        C = We1[i, 2 * HID + 1:]
        P, Q = _tc_pq(h, A, B)
        Gr, Gc = _sc_gather_pair(P, Q, row, col)
        ef = _tc_edge(Gr, Gc, radial, edge_attr, edge_mask,
                      wr[:, plane], C[:, plane], be1[i][plane].reshape(1, -1),
                      We2[i][plane, :].astype(jnp.bfloat16),
                      be2[i].reshape(1, -1))
        agg = _sc_scatter_sorted(ef, perm_pad, rowp_pad, bounds)
        h = _tc_node(h, agg, h0, Wn1[i, :HID], Wn1[i, HID:2 * HID],
                     Wn1[i, 2 * HID:], bn1[i].reshape(1, -1), Wn2[i],
                     bn2[i].reshape(1, -1))

    hg = _tc_dec_pool(h, node_mask, Wd1, bd1.reshape(1, -1), Wd2,
                      bd2.reshape(1, -1))
    pred = _tc_head(hg, Wg1, bg1.reshape(1, -1), Wg2, bg2.reshape(1, -1))
    return pred[:, 0]


# speedup vs baseline: 1.9120x; 1.3967x over previous
import functools

import jax
import jax.numpy as jnp
from jax import lax
from jax.experimental import pallas as pl
from jax.experimental.pallas import tpu as pltpu
from jax.experimental.pallas import tpu_sc as plsc

N = 10000
E = 320000
HID = 256
NPG = 50
NL = 4

NC = 2
NS = 16
NW = NC * NS

BI = 128
NBLK = E // BI
HALF = N // NC
STRIPE = 320
HALF_PAD = NS * STRIPE
DUMMY = HALF_PAD - 1


def _silu(x):
    return x * jax.nn.sigmoid(x)


def _sc_gather_pair(tp, tq, row, col):
    D = tp.shape[1]
    iters = (NBLK + NW - 1) // NW

    @functools.partial(
        pl.kernel,
        out_type=(jax.ShapeDtypeStruct((E, D), jnp.float32),
                  jax.ShapeDtypeStruct((E, D), jnp.float32)),
        scratch_types=[pltpu.VMEM((BI,), jnp.int32),
                       pltpu.VMEM((BI,), jnp.int32),
                       pltpu.VMEM((BI, D), jnp.float32),
                       pltpu.VMEM((BI, D), jnp.float32),
                       pltpu.SemaphoreType.DMA],
        mesh=plsc.VectorSubcoreMesh(core_axis_name="c", subcore_axis_name="s"),
    )
    def k(tp_hbm, tq_hbm, row_hbm, col_hbm, gr_hbm, gc_hbm,
          idxr, idxc, pbuf, qbuf, sem):
        wid = lax.axis_index("s") * NC + lax.axis_index("c")

        def body(t, carry):
            j = wid + NW * t

            @pl.when(j < NBLK)
            def _():
                base = j * BI
                pltpu.sync_copy(row_hbm.at[pl.ds(base, BI)], idxr)
                pltpu.sync_copy(col_hbm.at[pl.ds(base, BI)], idxc)
                c1 = pltpu.async_copy(tp_hbm.at[idxr], pbuf, sem)
                c2 = pltpu.async_copy(tq_hbm.at[idxc], qbuf, sem)
                c1.wait()
                c2.wait()
                pltpu.sync_copy(pbuf, gr_hbm.at[pl.ds(base, BI)])
                pltpu.sync_copy(qbuf, gc_hbm.at[pl.ds(base, BI)])

            return carry

        lax.fori_loop(0, iters, body, 0)

    return k(tp, tq, row, col)


SHARD = 320
LAST_SHARD = N - (NW - 1) * SHARD


def _sc_scatter_sorted(ef, perm_pad, rowp_pad, bounds):
    @functools.partial(
        pl.kernel,
        out_type=jax.ShapeDtypeStruct((N, HID), jnp.float32),
        scratch_types=[pltpu.VMEM((56,), jnp.int32),
                       pltpu.VMEM((BI,), jnp.int32),
                       pltpu.VMEM((BI + 16,), jnp.int32),
                       pltpu.VMEM((BI, HID), jnp.float32),
                       pltpu.VMEM((SHARD + 8, HID), jnp.float32),
                       pltpu.SemaphoreType.DMA],
        mesh=plsc.VectorSubcoreMesh(core_axis_name="c", subcore_axis_name="s"),
    )
    def k(ef_hbm, perm_hbm, rowp_hbm, bounds_hbm, out_hbm,
          bvuf, pbufi, rbuf, efb, acc, sem):
        wid = lax.axis_index("s") * NC + lax.axis_index("c")
        pltpu.sync_copy(bounds_hbm, bvuf.at[pl.ds(0, 40)])
        lo = bvuf[pl.ds(wid, 16)][0]
        hi = bvuf[pl.ds(wid + 1, 16)][0]
        base = wid * SHARD

        @functools.partial(plsc.parallel_loop, 0, SHARD + 8, unroll=4)
        def zr(r):
            for c in range(HID // 16):
                acc[r, pl.ds(c * 16, 16)] = jnp.zeros((16,), jnp.float32)

        jb0 = (lo // BI) * BI

        def body(t, carry):
            jb = pl.multiple_of(jb0 + t * BI, BI)

            @pl.when(jb < hi)
            def _():
                pltpu.sync_copy(perm_hbm.at[pl.ds(jb, BI)], pbufi)
                pltpu.sync_copy(rowp_hbm.at[pl.ds(jb, BI)], rbuf.at[pl.ds(0, BI)])
                pltpu.async_copy(ef_hbm.at[pbufi], efb, sem).wait()

                @functools.partial(plsc.parallel_loop, 0, BI, unroll=4)
                def rbody(k2):
                    rid = rbuf[pl.ds(k2, 16)][0]
                    loc = rid - base
                    ok = (jb + k2 < hi) & (loc >= 0) & (loc < SHARD)
                    loc = jnp.where(ok, loc, SHARD)
                    for c in range(HID // 16):
                        plsc.addupdate(acc.at[loc, pl.ds(c * 16, 16)],
                                       efb[k2, pl.ds(c * 16, 16)])

            return carry

        lax.fori_loop(0, NBLK, body, 0)

        @pl.when(wid < NW - 1)
        def _():
            pltpu.sync_copy(acc.at[pl.ds(0, SHARD)],
                            out_hbm.at[pl.ds(base, SHARD)])

        @pl.when(wid == NW - 1)
        def _():
            pltpu.sync_copy(acc.at[pl.ds(0, LAST_SHARD)],
                            out_hbm.at[pl.ds(base, LAST_SHARD)])

    return k(ef, perm_pad, rowp_pad, bounds)


def _tc_emb(h0, Wemb, bemb):
    BN = 2000

    def body(x_ref, w_ref, b_ref, o_ref):
        o_ref[...] = (jnp.dot(x_ref[...], w_ref[...],
                              preferred_element_type=jnp.float32) + b_ref[...])

    return pl.pallas_call(
        body,
        grid=(N // BN,),
        in_specs=[pl.BlockSpec((BN, h0.shape[1]), lambda i: (i, 0)),
                  pl.BlockSpec(Wemb.shape, lambda i: (0, 0)),
                  pl.BlockSpec((1, HID), lambda i: (0, 0))],
        out_specs=pl.BlockSpec((BN, HID), lambda i: (i, 0)),
        out_shape=jax.ShapeDtypeStruct((N, HID), jnp.float32),
    )(h0, Wemb, bemb)


def _tc_pq(h, A, B):
    BN = 1000

    def body(h_ref, a_ref, b_ref, p_ref, q_ref):
        hb = h_ref[...]
        p = jnp.dot(hb, a_ref[...], preferred_element_type=jnp.float32)
        q = jnp.dot(hb, b_ref[...], preferred_element_type=jnp.float32)
        p_ref[...] = p.astype(jnp.bfloat16)
        q_ref[...] = q.astype(jnp.bfloat16)

    P, Q = pl.pallas_call(
        body,
        grid=(N // BN,),
        in_specs=[pl.BlockSpec((BN, HID), lambda i: (i, 0)),
                  pl.BlockSpec((HID, HID), lambda i: (0, 0)),
                  pl.BlockSpec((HID, HID), lambda i: (0, 0))],
        out_specs=[pl.BlockSpec((BN, HID), lambda i: (i, 0)),
                   pl.BlockSpec((BN, HID), lambda i: (i, 0))],
        out_shape=[jax.ShapeDtypeStruct((N, HID), jnp.bfloat16),
                   jax.ShapeDtypeStruct((N, HID), jnp.bfloat16)],
    )(h, A, B)
    Ppk = jax.lax.bitcast_convert_type(P.reshape(N, HID // 2, 2), jnp.float32)
    Qpk = jax.lax.bitcast_convert_type(Q.reshape(N, HID // 2, 2), jnp.float32)
    return Ppk, Qpk


def _tc_radial(xr, xc):
    BE = 2000
    D = xr.shape[1]

    def body(r_ref, c_ref, o_ref):
        d = r_ref[...] - c_ref[...]
        o_ref[...] = jnp.sum(d * d, axis=1, keepdims=True)

    return pl.pallas_call(
        body,
        grid=(E // BE,),
        in_specs=[pl.BlockSpec((BE, D), lambda i: (i, 0)),
                  pl.BlockSpec((BE, D), lambda i: (i, 0))],
        out_specs=pl.BlockSpec((BE, 1), lambda i: (i, 0)),
        out_shape=jax.ShapeDtypeStruct((E, 1), jnp.float32),
    )(xr, xc)


def _unpack_planes(packed_f32):
    u = jax.lax.bitcast_convert_type(packed_f32, jnp.int32)
    even = jax.lax.bitcast_convert_type(u << 16, jnp.float32)
    odd = jax.lax.bitcast_convert_type(u & jnp.int32(-65536), jnp.float32)
    return jnp.concatenate([even, odd], axis=1)


def _tc_edge(Gr, Gc, radial, attr, mask, wr_p, C_p, be1_p, We2_p, be2):
    BE = 512

    def body(gr_ref, gc_ref, rad_ref, at_ref, mk_ref,
             wr_ref, c_ref, b1_ref, w2_ref, b2_ref, o_ref):
        t = (_unpack_planes(gr_ref[...]) + _unpack_planes(gc_ref[...])
             + rad_ref[...] * wr_ref[...] + b1_ref[...])
        at = at_ref[...]
        cw = c_ref[...]
        for k in range(4):
            t += at[:, k:k + 1] * cw[k:k + 1, :]
        m = _silu(t)
        z = jnp.dot(m.astype(jnp.bfloat16), w2_ref[...],
                    preferred_element_type=jnp.float32) + b2_ref[...]
        o_ref[...] = _silu(z) * mk_ref[...]

    return pl.pallas_call(
        body,
        grid=(E // BE,),
        in_specs=[pl.BlockSpec((BE, HID // 2), lambda i: (i, 0)),
                  pl.BlockSpec((BE, HID // 2), lambda i: (i, 0)),
                  pl.BlockSpec((BE, 1), lambda i: (i, 0)),
                  pl.BlockSpec((BE, 4), lambda i: (i, 0)),
                  pl.BlockSpec((BE, 1), lambda i: (i, 0)),
                  pl.BlockSpec((1, HID), lambda i: (0, 0)),
                  pl.BlockSpec((4, HID), lambda i: (0, 0)),
                  pl.BlockSpec((1, HID), lambda i: (0, 0)),
                  pl.BlockSpec((HID, HID), lambda i: (0, 0)),
                  pl.BlockSpec((1, HID), lambda i: (0, 0))],
        out_specs=pl.BlockSpec((BE, HID), lambda i: (i, 0)),
        out_shape=jax.ShapeDtypeStruct((E, HID), jnp.float32),
    )(Gr, Gc, radial, attr, mask, wr_p, C_p, be1_p, We2_p, be2)


def _tc_node(h, agg, h0, Wh, Wa, W0, bn1, Wn2, bn2):
    BN = 1000

    def body(h_ref, a_ref, h0_ref, wh_ref, wa_ref, w0_ref,
             b1_ref, w2_ref, b2_ref, o_ref):
        hb = h_ref[...]
        t = (jnp.dot(hb, wh_ref[...], preferred_element_type=jnp.float32)
             + jnp.dot(a_ref[...], wa_ref[...], preferred_element_type=jnp.float32)
             + jnp.dot(h0_ref[...], w0_ref[...], preferred_element_type=jnp.float32)
             + b1_ref[...])
        u = jnp.dot(_silu(t), w2_ref[...], preferred_element_type=jnp.float32) + b2_ref[...]
        o_ref[...] = hb + u

    return pl.pallas_call(
        body,
        grid=(N // BN,),
        in_specs=[pl.BlockSpec((BN, HID), lambda i: (i, 0)),
                  pl.BlockSpec((BN, HID), lambda i: (i, 0)),
                  pl.BlockSpec((BN, h0.shape[1]), lambda i: (i, 0)),
                  pl.BlockSpec((HID, HID), lambda i: (0, 0)),
                  pl.BlockSpec((HID, HID), lambda i: (0, 0)),
                  pl.BlockSpec((h0.shape[1], HID), lambda i: (0, 0)),
                  pl.BlockSpec((1, HID), lambda i: (0, 0)),
                  pl.BlockSpec((HID, HID), lambda i: (0, 0)),
                  pl.BlockSpec((1, HID), lambda i: (0, 0))],
        out_specs=pl.BlockSpec((BN, HID), lambda i: (i, 0)),
        out_shape=jax.ShapeDtypeStruct((N, HID), jnp.float32),
    )(h, agg, h0, Wh, Wa, W0, bn1, Wn2, bn2)


def _tc_dec_pool(h, mask, Wd1, bd1, Wd2, bd2):
    BN = 1000
    GPB = BN // NPG
    NG = N // NPG

    def body(h_ref, mk_ref, w1_ref, b1_ref, w2_ref, b2_ref, o_ref):
        t = _silu(jnp.dot(h_ref[...], w1_ref[...],
                          preferred_element_type=jnp.float32) + b1_ref[...])
        hd = (jnp.dot(t, w2_ref[...], preferred_element_type=jnp.float32)
              + b2_ref[...]) * mk_ref[...]
        for g in range(GPB):
            o_ref[0, g, :] = jnp.sum(hd[g * NPG:(g + 1) * NPG, :], axis=0)

    out = pl.pallas_call(
        body,
        grid=(N // BN,),
        in_specs=[pl.BlockSpec((BN, HID), lambda i: (i, 0)),
                  pl.BlockSpec((BN, 1), lambda i: (i, 0)),
                  pl.BlockSpec((HID, HID), lambda i: (0, 0)),
                  pl.BlockSpec((1, HID), lambda i: (0, 0)),
                  pl.BlockSpec((HID, HID), lambda i: (0, 0)),
                  pl.BlockSpec((1, HID), lambda i: (0, 0))],
        out_specs=pl.BlockSpec((1, GPB, HID), lambda i: (i, 0, 0)),
        out_shape=jax.ShapeDtypeStruct((N // BN, GPB, HID), jnp.float32),
    )(h, mask, Wd1, bd1, Wd2, bd2)
    return out.reshape(NG, HID)


def _tc_head(hg, Wg1, bg1, Wg2, bg2):
    NG = hg.shape[0]

    def body(h_ref, w1_ref, b1_ref, w2_ref, b2_ref, o_ref):
        t = _silu(jnp.dot(h_ref[...], w1_ref[...],
                          preferred_element_type=jnp.float32) + b1_ref[...])
        o_ref[...] = jnp.dot(t, w2_ref[...],
                             preferred_element_type=jnp.float32) + b2_ref[...]

    return pl.pallas_call(
        body,
        grid=(1,),
        in_specs=[pl.BlockSpec((NG, HID), lambda i: (0, 0)),
                  pl.BlockSpec((HID, HID), lambda i: (0, 0)),
                  pl.BlockSpec((1, HID), lambda i: (0, 0)),
                  pl.BlockSpec((HID, 1), lambda i: (0, 0)),
                  pl.BlockSpec((1, 1), lambda i: (0, 0))],
        out_specs=pl.BlockSpec((NG, 1), lambda i: (0, 0)),
        out_shape=jax.ShapeDtypeStruct((NG, 1), jnp.float32),
    )(hg, Wg1, bg1, Wg2, bg2)


def kernel(h0, x, edges, edge_attr, node_mask, edge_mask, n_nodes,
           Wemb, bemb, We1, be1, We2, be2, Wn1, bn1, Wn2, bn2,
           Wd1, bd1, Wd2, bd2, Wg1, bg1, Wg2, bg2):
    row = edges[0]
    col = edges[1]
    xpad = jnp.pad(x, ((0, 0), (0, 128 - x.shape[1])))

    order = jnp.argsort(row).astype(jnp.int32)
    rows_sorted = row[order]
    bounds = jnp.searchsorted(
        rows_sorted, jnp.arange(NW + 1, dtype=jnp.int32) * SHARD).astype(jnp.int32)
    bounds = jnp.pad(bounds, (0, 40 - NW - 1))
    perm_pad = jnp.pad(order, (0, BI))
    rowp_pad = jnp.pad(rows_sorted, (0, BI))

    h = _tc_emb(h0, Wemb, bemb.reshape(1, -1))
    xr, xc = _sc_gather_pair(xpad, xpad, row, col)
    radial = _tc_radial(xr, xc)

    plane = jnp.concatenate([jnp.arange(0, HID, 2), jnp.arange(1, HID, 2)])

    for i in range(NL):
        A = We1[i, :HID]
        B = We1[i, HID:2 * HID]
        wr = We1[i, 2 * HID:2 * HID + 1]
        C = We1[i, 2 * HID + 1:]
        P, Q = _tc_pq(h, A, B)
        Gr, Gc = _sc_gather_pair(P, Q, row, col)
        ef = _tc_edge(Gr, Gc, radial, edge_attr, edge_mask,
                      wr[:, plane], C[:, plane], be1[i][plane].reshape(1, -1),
                      We2[i][plane, :].astype(jnp.bfloat16),
                      be2[i].reshape(1, -1))
        agg = _sc_scatter_sorted(ef, perm_pad, rowp_pad, bounds)
        h = _tc_node(h, agg, h0, Wn1[i, :HID], Wn1[i, HID:2 * HID],
                     Wn1[i, 2 * HID:], bn1[i].reshape(1, -1), Wn2[i],
                     bn2[i].reshape(1, -1))

    hg = _tc_dec_pool(h, node_mask, Wd1, bd1.reshape(1, -1), Wd2,
                      bd2.reshape(1, -1))
    pred = _tc_head(hg, Wg1, bg1.reshape(1, -1), Wg2, bg2.reshape(1, -1))
    return pred[:, 0]
